# baseline (speedup 1.0000x reference)
import functools

import jax
import jax.numpy as jnp
from jax import lax
from jax.experimental import pallas as pl
from jax.experimental.pallas import tpu as pltpu
from jax.experimental.pallas import tpu_sc as plsc

N = 10000
D = 128
G = 64
L = 3
BN_EPS = 1e-5

NC = 2
NS = 16
NW = NC * NS
CHUNK = 128
BLK = 1000
NBLK = N // BLK
N_ACC = 10240
ROWS_PER_TILE = N_ACC // NS
ZCOPY = 128
CPW0 = 104
CPW1 = 56
C1_OFF = NS * CPW0
DEAD = N


def _make_agg():
    mesh = plsc.VectorSubcoreMesh(core_axis_name="c", subcore_axis_name="s")

    @functools.partial(
        pl.kernel,
        out_type=jax.ShapeDtypeStruct((NC, N_ACC, D), jnp.float32),
        mesh=mesh,
        scratch_types=[
            pltpu.VMEM((CPW0, CHUNK), jnp.int32),
            pltpu.VMEM((CPW0, CHUNK), jnp.int32),
            pltpu.VMEM((CHUNK, D), jnp.float32),
            pltpu.VMEM_SHARED((N_ACC, D), jnp.float32),
            pltpu.SemaphoreType.DMA,
        ],
    )
    def agg(z_hbm, src_hbm, dst_hbm, out_hbm, src_v, dst_v, b0, acc_sh, gsem):
        c = lax.axis_index("c")
        s = lax.axis_index("s")
        n_c = lax.select(c == 0, jnp.int32(CPW0), jnp.int32(CPW1))
        cb = lax.select(c == 0, s * CPW0, C1_OFF + s * CPW1)

        def zrow(r, carry):
            for k in range(D // 16):
                b0[r, pl.ds(k * 16, 16)] = jnp.zeros((16,), jnp.float32)
            return carry
        lax.fori_loop(0, CHUNK, zrow, 0)
        base = s * ROWS_PER_TILE
        for k in range(ROWS_PER_TILE // ZCOPY):
            pltpu.sync_copy(b0.at[pl.ds(0, ZCOPY)],
                            acc_sh.at[pl.ds(base + k * ZCOPY, ZCOPY)])
        plsc.subcore_barrier()

        pltpu.sync_copy(src_hbm.at[pl.ds(cb, CPW1)], src_v.at[pl.ds(0, CPW1)])
        pltpu.sync_copy(dst_hbm.at[pl.ds(cb, CPW1)], dst_v.at[pl.ds(0, CPW1)])

        @pl.when(c == 0)
        def _():
            pltpu.sync_copy(src_hbm.at[pl.ds(cb + CPW1, CPW0 - CPW1)],
                            src_v.at[pl.ds(CPW1, CPW0 - CPW1)])
            pltpu.sync_copy(dst_hbm.at[pl.ds(cb + CPW1, CPW0 - CPW1)],
                            dst_v.at[pl.ds(CPW1, CPW0 - CPW1)])

        def body(j, carry):
            pltpu.async_copy(z_hbm.at[src_v.at[j]], b0, gsem).wait()
            pltpu.sync_copy(b0, acc_sh.at[dst_v.at[j]], add=True)
            return carry
        lax.fori_loop(0, n_c, body, 0)
        plsc.subcore_barrier()

        for k in range(ROWS_PER_TILE // ZCOPY):
            off = base + k * ZCOPY
            pltpu.sync_copy(acc_sh.at[pl.ds(off, ZCOPY)],
                            b0.at[pl.ds(0, ZCOPY)])
            pltpu.sync_copy(b0.at[pl.ds(0, ZCOPY)],
                            out_hbm.at[c, pl.ds(off, ZCOPY)])

    return agg


def _mlp_body(last: bool, z_ref, parts_ref, bt_ref, w1_ref, b1_ref, w2_ref,
              b2_ref, h_ref, g_ref):
    h = z_ref[...] + parts_ref[0] + parts_ref[1]
    h = jnp.maximum(
        jnp.dot(h, w1_ref[...], preferred_element_type=jnp.float32) + b1_ref[...],
        0.0)
    h = jnp.dot(h, w2_ref[...], preferred_element_type=jnp.float32) + b2_ref[...]
    if not last:
        h = jnp.maximum(h, 0.0)
    h_ref[...] = h

    b = bt_ref[0, 0, :]
    oh_t = (lax.broadcasted_iota(jnp.int32, (G, BLK), 0) == b[None, :]
            ).astype(jnp.float32)
    gpart = jnp.dot(oh_t, h, preferred_element_type=jnp.float32)

    @pl.when(pl.program_id(0) == 0)
    def _():
        g_ref[...] = jnp.zeros_like(g_ref)
    g_ref[...] += gpart


def _make_mlp(last: bool):
    return pl.pallas_call(
        functools.partial(_mlp_body, last),
        grid=(NBLK,),
        in_specs=[
            pl.BlockSpec((BLK, D), lambda i: (i, 0)),
            pl.BlockSpec((NC, BLK, D), lambda i: (0, i, 0)),
            pl.BlockSpec((1, 1, BLK), lambda i: (i, 0, 0)),
            pl.BlockSpec((D, D), lambda i: (0, 0)),
            pl.BlockSpec((1, D), lambda i: (0, 0)),
            pl.BlockSpec((D, D), lambda i: (0, 0)),
            pl.BlockSpec((1, D), lambda i: (0, 0)),
        ],
        out_specs=[
            pl.BlockSpec((BLK, D), lambda i: (i, 0)),
            pl.BlockSpec((G, D), lambda i: (0, 0)),
        ],
        out_shape=[
            jax.ShapeDtypeStruct((N, D), jnp.float32),
            jax.ShapeDtypeStruct((G, D), jnp.float32),
        ],
    )


def kernel(x, edge_index, batch,
           W1_0, b1_0, W2_0, b2_0, gamma_0, beta_0,
           W1_1, b1_1, W2_1, b2_1, gamma_1, beta_1,
           W1_2, b1_2, W2_2, b2_2, gamma_2, beta_2):
    params = [
        (W1_0, b1_0, W2_0, b2_0, gamma_0, beta_0),
        (W1_1, b1_1, W2_1, b2_1, gamma_1, beta_1),
        (W1_2, b1_2, W2_2, b2_2, gamma_2, beta_2),
    ]
    e = edge_index.shape[1]
    e_pad = NS * (CPW0 + CPW1) * CHUNK
    assert e_pad >= e
    pad_cols = jnp.broadcast_to(
        jnp.array([[0], [DEAD]], jnp.int32), (2, e_pad - e))
    ei = jnp.concatenate([edge_index, pad_cols], axis=1)
    ei = ei.reshape(2, NS * (CPW0 + CPW1), CHUNK)
    src_w = ei[0]
    dst_w = ei[1]
    batch3 = batch.reshape(NBLK, 1, BLK)

    agg_fn = _make_agg()
    mlp_mid = _make_mlp(last=False)
    mlp_last = _make_mlp(last=True)

    z = x
    zs, gs = [], []
    for l in range(L):
        W1, b1, W2, b2, gamma, beta = params[l]
        scale = gamma / jnp.sqrt(1.0 + BN_EPS)
        w2f = W2 * scale[None, :]
        b2f = (b2 * scale + beta).reshape(1, D)
        b1r = b1.reshape(1, D)

        parts = agg_fn(z, src_w, dst_w)
        mlp = mlp_last if l == L - 1 else mlp_mid
        h, g = mlp(z, parts, batch3, W1, b1r, w2f, b2f)
        zs.append(h)
        gs.append(g)
        z = h

    return (jnp.concatenate(zs, axis=1), jnp.concatenate(gs, axis=1))


# speedup vs baseline: 1.8429x; 1.8429x over previous
import functools

import jax
import jax.numpy as jnp
from jax import lax
from jax.experimental import pallas as pl
from jax.experimental.pallas import tpu as pltpu
from jax.experimental.pallas import tpu_sc as plsc

N = 10000
D = 128
G = 64
L = 3
BN_EPS = 1e-5

NC = 2
NS = 16
NW = NC * NS
CHUNK = 128
BLK = 1000
NBLK = N // BLK
N_ACC = 10240
ROWS_PER_TILE = N_ACC // NS
ZCOPY = 128
CPW0 = 105
CPW1 = 52
DEAD = N


def _make_agg():
    mesh = plsc.VectorSubcoreMesh(core_axis_name="c", subcore_axis_name="s")

    @functools.partial(
        pl.kernel,
        out_type=jax.ShapeDtypeStruct((NC, N_ACC, D), jnp.float32),
        mesh=mesh,
        scratch_types=[
            pltpu.VMEM((CPW0, CHUNK), jnp.int32),
            pltpu.VMEM((CPW0, CHUNK), jnp.int32),
            pltpu.VMEM((CHUNK, D), jnp.float32),
            pltpu.VMEM_SHARED((N_ACC, D), jnp.float32),
            pltpu.SemaphoreType.DMA,
        ],
    )
    def agg(z_hbm, src_hbm, dst_hbm, out_hbm, src_v, dst_v, b0, acc_sh, gsem):
        c = lax.axis_index("c")
        s = lax.axis_index("s")
        row = c * NS + s
        n_c = lax.select(c == 0, jnp.int32(CPW0), jnp.int32(CPW1))

        def zrow(r, carry):
            for k in range(D // 16):
                b0[r, pl.ds(k * 16, 16)] = jnp.zeros((16,), jnp.float32)
            return carry
        lax.fori_loop(0, CHUNK, zrow, 0)
        base = s * ROWS_PER_TILE
        for k in range(ROWS_PER_TILE // ZCOPY):
            pltpu.sync_copy(b0.at[pl.ds(0, ZCOPY)],
                            acc_sh.at[pl.ds(base + k * ZCOPY, ZCOPY)])
        plsc.subcore_barrier()

        pltpu.sync_copy(src_hbm.at[row], src_v)
        pltpu.sync_copy(dst_hbm.at[row], dst_v)

        def body(j, carry):
            pltpu.async_copy(z_hbm.at[src_v.at[j]], b0, gsem).wait()
            pltpu.sync_copy(b0, acc_sh.at[dst_v.at[j]], add=True)
            return carry
        lax.fori_loop(0, n_c, body, 0)
        plsc.subcore_barrier()

        for k in range(ROWS_PER_TILE // ZCOPY):
            off = base + k * ZCOPY
            pltpu.sync_copy(acc_sh.at[pl.ds(off, ZCOPY)],
                            b0.at[pl.ds(0, ZCOPY)])
            pltpu.sync_copy(b0.at[pl.ds(0, ZCOPY)],
                            out_hbm.at[c, pl.ds(off, ZCOPY)])

    return agg


def _mlp_body(last: bool, z_ref, parts_ref, bt_ref, w1_ref, b1_ref, w2_ref,
              b2_ref, h_ref, g_ref):
    h = z_ref[...] + parts_ref[0] + parts_ref[1]
    h = jnp.maximum(
        jnp.dot(h, w1_ref[...], preferred_element_type=jnp.float32) + b1_ref[...],
        0.0)
    h = jnp.dot(h, w2_ref[...], preferred_element_type=jnp.float32) + b2_ref[...]
    if not last:
        h = jnp.maximum(h, 0.0)
    h_ref[...] = h

    b = bt_ref[0, 0, :]
    oh_t = (lax.broadcasted_iota(jnp.int32, (G, BLK), 0) == b[None, :]
            ).astype(jnp.float32)
    gpart = jnp.dot(oh_t, h, preferred_element_type=jnp.float32)

    @pl.when(pl.program_id(0) == 0)
    def _():
        g_ref[...] = jnp.zeros_like(g_ref)
    g_ref[...] += gpart


def _make_mlp(last: bool):
    return pl.pallas_call(
        functools.partial(_mlp_body, last),
        grid=(NBLK,),
        in_specs=[
            pl.BlockSpec((BLK, D), lambda i: (i, 0)),
            pl.BlockSpec((NC, BLK, D), lambda i: (0, i, 0)),
            pl.BlockSpec((1, 1, BLK), lambda i: (i, 0, 0)),
            pl.BlockSpec((D, D), lambda i: (0, 0)),
            pl.BlockSpec((1, D), lambda i: (0, 0)),
            pl.BlockSpec((D, D), lambda i: (0, 0)),
            pl.BlockSpec((1, D), lambda i: (0, 0)),
        ],
        out_specs=[
            pl.BlockSpec((BLK, D), lambda i: (i, 0)),
            pl.BlockSpec((G, D), lambda i: (0, 0)),
        ],
        out_shape=[
            jax.ShapeDtypeStruct((N, D), jnp.float32),
            jax.ShapeDtypeStruct((G, D), jnp.float32),
        ],
    )


def kernel(x, edge_index, batch,
           W1_0, b1_0, W2_0, b2_0, gamma_0, beta_0,
           W1_1, b1_1, W2_1, b2_1, gamma_1, beta_1,
           W1_2, b1_2, W2_2, b2_2, gamma_2, beta_2):
    params = [
        (W1_0, b1_0, W2_0, b2_0, gamma_0, beta_0),
        (W1_1, b1_1, W2_1, b2_1, gamma_1, beta_1),
        (W1_2, b1_2, W2_2, b2_2, gamma_2, beta_2),
    ]
    e = edge_index.shape[1]
    e_pad = NS * (CPW0 + CPW1) * CHUNK
    assert e_pad >= e
    pad_cols = jnp.broadcast_to(
        jnp.array([[0], [DEAD]], jnp.int32), (2, e_pad - e))
    ei = jnp.concatenate([edge_index, pad_cols], axis=1)
    split = NS * CPW0 * CHUNK
    slot_pad = jnp.zeros((NS, CPW0 - CPW1, CHUNK), jnp.int32)

    def pools(flat):
        p0 = flat[:split].reshape(NS, CPW0, CHUNK)
        p1 = flat[split:].reshape(NS, CPW1, CHUNK)
        return jnp.concatenate(
            [p0, jnp.concatenate([p1, slot_pad], axis=1)], axis=0)
    src_w = pools(ei[0])
    dst_w = pools(ei[1])
    batch3 = batch.reshape(NBLK, 1, BLK)

    agg_fn = _make_agg()
    mlp_mid = _make_mlp(last=False)
    mlp_last = _make_mlp(last=True)

    z = x
    zs, gs = [], []
    for l in range(L):
        W1, b1, W2, b2, gamma, beta = params[l]
        scale = gamma / jnp.sqrt(1.0 + BN_EPS)
        w2f = W2 * scale[None, :]
        b2f = (b2 * scale + beta).reshape(1, D)
        b1r = b1.reshape(1, D)

        parts = agg_fn(z, src_w, dst_w)
        mlp = mlp_last if l == L - 1 else mlp_mid
        h, g = mlp(z, parts, batch3, W1, b1r, w2f, b2f)
        zs.append(h)
        gs.append(g)
        z = h

    return (jnp.concatenate(zs, axis=1), jnp.concatenate(gs, axis=1))


# speedup vs baseline: 1.8468x; 1.0021x over previous
import functools

import jax
import jax.numpy as jnp
from jax import lax
from jax.experimental import pallas as pl
from jax.experimental.pallas import tpu as pltpu
from jax.experimental.pallas import tpu_sc as plsc

N = 10000
D = 128
G = 64
L = 3
BN_EPS = 1e-5

NC = 2
NS = 16
NW = NC * NS
CHUNK = 128
BLK = 1000
NBLK = N // BLK
N_ACC = 10240
ROWS_PER_TILE = N_ACC // NS
ZCOPY = 128
CPW0 = 105
CPW1 = 52
DEAD = N


def _make_agg():
    mesh = plsc.VectorSubcoreMesh(core_axis_name="c", subcore_axis_name="s")

    @functools.partial(
        pl.kernel,
        out_type=jax.ShapeDtypeStruct((NC, N_ACC, D), jnp.float32),
        mesh=mesh,
        scratch_types=[
            pltpu.VMEM((CPW0, CHUNK), jnp.int32),
            pltpu.VMEM((CPW0, CHUNK), jnp.int32),
            pltpu.VMEM((CHUNK, D), jnp.float32),
            pltpu.VMEM_SHARED((N_ACC, D), jnp.float32),
            pltpu.SemaphoreType.DMA,
        ],
    )
    def agg(z_hbm, src_hbm, dst_hbm, out_hbm, src_v, dst_v, b0, acc_sh, gsem):
        c = lax.axis_index("c")
        s = lax.axis_index("s")
        row = c * NS + s
        n_c = lax.select(c == 0, jnp.int32(CPW0), jnp.int32(CPW1))

        def zrow(r, carry):
            for k in range(D // 16):
                b0[r, pl.ds(k * 16, 16)] = jnp.zeros((16,), jnp.float32)
            return carry
        lax.fori_loop(0, CHUNK, zrow, 0)
        base = s * ROWS_PER_TILE
        for k in range(ROWS_PER_TILE // ZCOPY):
            pltpu.sync_copy(b0.at[pl.ds(0, ZCOPY)],
                            acc_sh.at[pl.ds(base + k * ZCOPY, ZCOPY)])
        plsc.subcore_barrier()

        pltpu.sync_copy(src_hbm.at[row], src_v)
        pltpu.sync_copy(dst_hbm.at[row], dst_v)

        def body(j, carry):
            pltpu.async_copy(z_hbm.at[src_v.at[j]], b0, gsem).wait()
            pltpu.sync_copy(b0, acc_sh.at[dst_v.at[j]], add=True)
            return carry
        lax.fori_loop(0, n_c, body, 0)
        plsc.subcore_barrier()

        pltpu.sync_copy(acc_sh.at[pl.ds(base, ROWS_PER_TILE)],
                        out_hbm.at[c, pl.ds(base, ROWS_PER_TILE)])

    return agg


def _mlp_body(last: bool, z_ref, parts_ref, bt_ref, w1_ref, b1_ref, w2_ref,
              b2_ref, h_ref, g_ref):
    h = z_ref[...] + parts_ref[0] + parts_ref[1]
    h = jnp.maximum(
        jnp.dot(h, w1_ref[...], preferred_element_type=jnp.float32) + b1_ref[...],
        0.0)
    h = jnp.dot(h, w2_ref[...], preferred_element_type=jnp.float32) + b2_ref[...]
    if not last:
        h = jnp.maximum(h, 0.0)
    h_ref[...] = h

    b = bt_ref[0, 0, :]
    oh_t = (lax.broadcasted_iota(jnp.int32, (G, BLK), 0) == b[None, :]
            ).astype(jnp.float32)
    gpart = jnp.dot(oh_t, h, preferred_element_type=jnp.float32)

    @pl.when(pl.program_id(0) == 0)
    def _():
        g_ref[...] = jnp.zeros_like(g_ref)
    g_ref[...] += gpart


def _make_mlp(last: bool):
    return pl.pallas_call(
        functools.partial(_mlp_body, last),
        grid=(NBLK,),
        in_specs=[
            pl.BlockSpec((BLK, D), lambda i: (i, 0)),
            pl.BlockSpec((NC, BLK, D), lambda i: (0, i, 0)),
            pl.BlockSpec((1, 1, BLK), lambda i: (i, 0, 0)),
            pl.BlockSpec((D, D), lambda i: (0, 0)),
            pl.BlockSpec((1, D), lambda i: (0, 0)),
            pl.BlockSpec((D, D), lambda i: (0, 0)),
            pl.BlockSpec((1, D), lambda i: (0, 0)),
        ],
        out_specs=[
            pl.BlockSpec((BLK, D), lambda i: (i, 0)),
            pl.BlockSpec((G, D), lambda i: (0, 0)),
        ],
        out_shape=[
            jax.ShapeDtypeStruct((N, D), jnp.float32),
            jax.ShapeDtypeStruct((G, D), jnp.float32),
        ],
    )


def kernel(x, edge_index, batch,
           W1_0, b1_0, W2_0, b2_0, gamma_0, beta_0,
           W1_1, b1_1, W2_1, b2_1, gamma_1, beta_1,
           W1_2, b1_2, W2_2, b2_2, gamma_2, beta_2):
    params = [
        (W1_0, b1_0, W2_0, b2_0, gamma_0, beta_0),
        (W1_1, b1_1, W2_1, b2_1, gamma_1, beta_1),
        (W1_2, b1_2, W2_2, b2_2, gamma_2, beta_2),
    ]
    e = edge_index.shape[1]
    e_pad = NS * (CPW0 + CPW1) * CHUNK
    assert e_pad >= e
    pad_cols = jnp.broadcast_to(
        jnp.array([[0], [DEAD]], jnp.int32), (2, e_pad - e))
    ei = jnp.concatenate([edge_index, pad_cols], axis=1)
    split = NS * CPW0 * CHUNK
    slot_pad = jnp.zeros((NS, CPW0 - CPW1, CHUNK), jnp.int32)

    def pools(flat):
        p0 = flat[:split].reshape(NS, CPW0, CHUNK)
        p1 = flat[split:].reshape(NS, CPW1, CHUNK)
        return jnp.concatenate(
            [p0, jnp.concatenate([p1, slot_pad], axis=1)], axis=0)
    src_w = pools(ei[0])
    dst_w = pools(ei[1])
    batch3 = batch.reshape(NBLK, 1, BLK)

    agg_fn = _make_agg()
    mlp_mid = _make_mlp(last=False)
    mlp_last = _make_mlp(last=True)

    z = x
    zs, gs = [], []
    for l in range(L):
        W1, b1, W2, b2, gamma, beta = params[l]
        scale = gamma / jnp.sqrt(1.0 + BN_EPS)
        w2f = W2 * scale[None, :]
        b2f = (b2 * scale + beta).reshape(1, D)
        b1r = b1.reshape(1, D)

        parts = agg_fn(z, src_w, dst_w)
        mlp = mlp_last if l == L - 1 else mlp_mid
        h, g = mlp(z, parts, batch3, W1, b1r, w2f, b2f)
        zs.append(h)
        gs.append(g)
        z = h

    return (jnp.concatenate(zs, axis=1), jnp.concatenate(gs, axis=1))


# speedup vs baseline: 1.8690x; 1.0120x over previous
import functools

import jax
import jax.numpy as jnp
from jax import lax
from jax.experimental import pallas as pl
from jax.experimental.pallas import tpu as pltpu
from jax.experimental.pallas import tpu_sc as plsc

N = 10000
D = 128
G = 64
L = 3
BN_EPS = 1e-5

NC = 2
NS = 16
NW = NC * NS
CHUNK = 128
BLK = 1000
NBLK = N // BLK
N_ACC = 10240
ROWS_PER_TILE = N_ACC // NS
ZCOPY = 128
CPW0 = 103
CPW1 = 54
DEAD = N


def _make_agg():
    mesh = plsc.VectorSubcoreMesh(core_axis_name="c", subcore_axis_name="s")

    @functools.partial(
        pl.kernel,
        out_type=jax.ShapeDtypeStruct((NC, N_ACC, D), jnp.float32),
        mesh=mesh,
        scratch_types=[
            pltpu.VMEM((CPW0, CHUNK), jnp.int32),
            pltpu.VMEM((CPW0, CHUNK), jnp.int32),
            pltpu.VMEM((CHUNK, D), jnp.float32),
            pltpu.VMEM_SHARED((N_ACC, D), jnp.float32),
            pltpu.SemaphoreType.DMA,
        ],
    )
    def agg(z_hbm, src_hbm, dst_hbm, out_hbm, src_v, dst_v, b0, acc_sh, gsem):
        c = lax.axis_index("c")
        s = lax.axis_index("s")
        row = c * NS + s
        n_c = lax.select(c == 0, jnp.int32(CPW0), jnp.int32(CPW1))

        def zrow(r, carry):
            for k in range(D // 16):
                b0[r, pl.ds(k * 16, 16)] = jnp.zeros((16,), jnp.float32)
            return carry
        lax.fori_loop(0, CHUNK, zrow, 0)
        base = s * ROWS_PER_TILE
        for k in range(ROWS_PER_TILE // ZCOPY):
            pltpu.sync_copy(b0.at[pl.ds(0, ZCOPY)],
                            acc_sh.at[pl.ds(base + k * ZCOPY, ZCOPY)])
        plsc.subcore_barrier()

        pltpu.sync_copy(src_hbm.at[row], src_v)
        pltpu.sync_copy(dst_hbm.at[row], dst_v)

        def body(j, carry):
            pltpu.async_copy(z_hbm.at[src_v.at[j]], b0, gsem).wait()
            pltpu.sync_copy(b0, acc_sh.at[dst_v.at[j]], add=True)
            return carry
        lax.fori_loop(0, n_c, body, 0)
        plsc.subcore_barrier()

        pltpu.sync_copy(acc_sh.at[pl.ds(base, ROWS_PER_TILE)],
                        out_hbm.at[c, pl.ds(base, ROWS_PER_TILE)])

    return agg


def _mlp_body(last: bool, z_ref, parts_ref, bt_ref, w1_ref, b1_ref, w2_ref,
              b2_ref, h_ref, g_ref):
    h = z_ref[...] + parts_ref[0] + parts_ref[1]
    h = jnp.maximum(
        jnp.dot(h, w1_ref[...], preferred_element_type=jnp.float32) + b1_ref[...],
        0.0)
    h = jnp.dot(h, w2_ref[...], preferred_element_type=jnp.float32) + b2_ref[...]
    if not last:
        h = jnp.maximum(h, 0.0)
    h_ref[...] = h

    b = bt_ref[0, 0, :]
    oh_t = (lax.broadcasted_iota(jnp.int32, (G, BLK), 0) == b[None, :]
            ).astype(jnp.float32)
    gpart = jnp.dot(oh_t, h, preferred_element_type=jnp.float32)

    @pl.when(pl.program_id(0) == 0)
    def _():
        g_ref[...] = jnp.zeros_like(g_ref)
    g_ref[...] += gpart


def _make_mlp(last: bool):
    return pl.pallas_call(
        functools.partial(_mlp_body, last),
        grid=(NBLK,),
        in_specs=[
            pl.BlockSpec((BLK, D), lambda i: (i, 0)),
            pl.BlockSpec((NC, BLK, D), lambda i: (0, i, 0)),
            pl.BlockSpec((1, 1, BLK), lambda i: (i, 0, 0)),
            pl.BlockSpec((D, D), lambda i: (0, 0)),
            pl.BlockSpec((1, D), lambda i: (0, 0)),
            pl.BlockSpec((D, D), lambda i: (0, 0)),
            pl.BlockSpec((1, D), lambda i: (0, 0)),
        ],
        out_specs=[
            pl.BlockSpec((BLK, D), lambda i: (i, 0)),
            pl.BlockSpec((G, D), lambda i: (0, 0)),
        ],
        out_shape=[
            jax.ShapeDtypeStruct((N, D), jnp.float32),
            jax.ShapeDtypeStruct((G, D), jnp.float32),
        ],
    )


def kernel(x, edge_index, batch,
           W1_0, b1_0, W2_0, b2_0, gamma_0, beta_0,
           W1_1, b1_1, W2_1, b2_1, gamma_1, beta_1,
           W1_2, b1_2, W2_2, b2_2, gamma_2, beta_2):
    params = [
        (W1_0, b1_0, W2_0, b2_0, gamma_0, beta_0),
        (W1_1, b1_1, W2_1, b2_1, gamma_1, beta_1),
        (W1_2, b1_2, W2_2, b2_2, gamma_2, beta_2),
    ]
    e = edge_index.shape[1]
    e_pad = NS * (CPW0 + CPW1) * CHUNK
    assert e_pad >= e
    pad_cols = jnp.broadcast_to(
        jnp.array([[0], [DEAD]], jnp.int32), (2, e_pad - e))
    ei = jnp.concatenate([edge_index, pad_cols], axis=1)
    split = NS * CPW0 * CHUNK
    slot_pad = jnp.zeros((NS, CPW0 - CPW1, CHUNK), jnp.int32)

    def pools(flat):
        p0 = flat[:split].reshape(NS, CPW0, CHUNK)
        p1 = flat[split:].reshape(NS, CPW1, CHUNK)
        return jnp.concatenate(
            [p0, jnp.concatenate([p1, slot_pad], axis=1)], axis=0)
    src_w = pools(ei[0])
    dst_w = pools(ei[1])
    batch3 = batch.reshape(NBLK, 1, BLK)

    agg_fn = _make_agg()
    mlp_mid = _make_mlp(last=False)
    mlp_last = _make_mlp(last=True)

    z = x
    zs, gs = [], []
    for l in range(L):
        W1, b1, W2, b2, gamma, beta = params[l]
        scale = gamma / jnp.sqrt(1.0 + BN_EPS)
        w2f = W2 * scale[None, :]
        b2f = (b2 * scale + beta).reshape(1, D)
        b1r = b1.reshape(1, D)

        parts = agg_fn(z, src_w, dst_w)
        mlp = mlp_last if l == L - 1 else mlp_mid
        h, g = mlp(z, parts, batch3, W1, b1r, w2f, b2f)
        zs.append(h)
        gs.append(g)
        z = h

    return (jnp.concatenate(zs, axis=1), jnp.concatenate(gs, axis=1))


# speedup vs baseline: 1.9015x; 1.0174x over previous
import functools

import jax
import jax.numpy as jnp
from jax import lax
from jax.experimental import pallas as pl
from jax.experimental.pallas import tpu as pltpu
from jax.experimental.pallas import tpu_sc as plsc

N = 10000
D = 128
G = 64
L = 3
BN_EPS = 1e-5

NC = 2
NS = 16
NW = NC * NS
CHUNK = 128
BLK = 1000
NBLK = N // BLK
N_ACC = 10240
ROWS_PER_TILE = N_ACC // NS
ZCOPY = 128
CPW0 = 101
CPW1 = 56
DEAD = N


def _make_agg():
    mesh = plsc.VectorSubcoreMesh(core_axis_name="c", subcore_axis_name="s")

    @functools.partial(
        pl.kernel,
        out_type=jax.ShapeDtypeStruct((NC, N_ACC, D), jnp.float32),
        mesh=mesh,
        scratch_types=[
            pltpu.VMEM((CPW0, CHUNK), jnp.int32),
            pltpu.VMEM((CPW0, CHUNK), jnp.int32),
            pltpu.VMEM((CHUNK, D), jnp.float32),
            pltpu.VMEM_SHARED((N_ACC, D), jnp.float32),
            pltpu.SemaphoreType.DMA,
        ],
    )
    def agg(z_hbm, src_hbm, dst_hbm, out_hbm, src_v, dst_v, b0, acc_sh, gsem):
        c = lax.axis_index("c")
        s = lax.axis_index("s")
        row = c * NS + s
        n_c = lax.select(c == 0, jnp.int32(CPW0), jnp.int32(CPW1))

        def zrow(r, carry):
            for k in range(D // 16):
                b0[r, pl.ds(k * 16, 16)] = jnp.zeros((16,), jnp.float32)
            return carry
        lax.fori_loop(0, CHUNK, zrow, 0)
        base = s * ROWS_PER_TILE
        for k in range(ROWS_PER_TILE // ZCOPY):
            pltpu.sync_copy(b0.at[pl.ds(0, ZCOPY)],
                            acc_sh.at[pl.ds(base + k * ZCOPY, ZCOPY)])
        plsc.subcore_barrier()

        pltpu.sync_copy(src_hbm.at[row], src_v)
        pltpu.sync_copy(dst_hbm.at[row], dst_v)

        def body(j, carry):
            pltpu.async_copy(z_hbm.at[src_v.at[j]], b0, gsem).wait()
            pltpu.sync_copy(b0, acc_sh.at[dst_v.at[j]], add=True)
            return carry
        lax.fori_loop(0, n_c, body, 0)
        plsc.subcore_barrier()

        pltpu.sync_copy(acc_sh.at[pl.ds(base, ROWS_PER_TILE)],
                        out_hbm.at[c, pl.ds(base, ROWS_PER_TILE)])

    return agg


def _mlp_body(last: bool, z_ref, parts_ref, bt_ref, w1_ref, b1_ref, w2_ref,
              b2_ref, h_ref, g_ref):
    h = z_ref[...] + parts_ref[0] + parts_ref[1]
    h = jnp.maximum(
        jnp.dot(h, w1_ref[...], preferred_element_type=jnp.float32) + b1_ref[...],
        0.0)
    h = jnp.dot(h, w2_ref[...], preferred_element_type=jnp.float32) + b2_ref[...]
    if not last:
        h = jnp.maximum(h, 0.0)
    h_ref[...] = h

    b = bt_ref[0, 0, :]
    oh_t = (lax.broadcasted_iota(jnp.int32, (G, BLK), 0) == b[None, :]
            ).astype(jnp.float32)
    gpart = jnp.dot(oh_t, h, preferred_element_type=jnp.float32)

    @pl.when(pl.program_id(0) == 0)
    def _():
        g_ref[...] = jnp.zeros_like(g_ref)
    g_ref[...] += gpart


def _make_mlp(last: bool):
    return pl.pallas_call(
        functools.partial(_mlp_body, last),
        grid=(NBLK,),
        in_specs=[
            pl.BlockSpec((BLK, D), lambda i: (i, 0)),
            pl.BlockSpec((NC, BLK, D), lambda i: (0, i, 0)),
            pl.BlockSpec((1, 1, BLK), lambda i: (i, 0, 0)),
            pl.BlockSpec((D, D), lambda i: (0, 0)),
            pl.BlockSpec((1, D), lambda i: (0, 0)),
            pl.BlockSpec((D, D), lambda i: (0, 0)),
            pl.BlockSpec((1, D), lambda i: (0, 0)),
        ],
        out_specs=[
            pl.BlockSpec((BLK, D), lambda i: (i, 0)),
            pl.BlockSpec((G, D), lambda i: (0, 0)),
        ],
        out_shape=[
            jax.ShapeDtypeStruct((N, D), jnp.float32),
            jax.ShapeDtypeStruct((G, D), jnp.float32),
        ],
    )


def kernel(x, edge_index, batch,
           W1_0, b1_0, W2_0, b2_0, gamma_0, beta_0,
           W1_1, b1_1, W2_1, b2_1, gamma_1, beta_1,
           W1_2, b1_2, W2_2, b2_2, gamma_2, beta_2):
    params = [
        (W1_0, b1_0, W2_0, b2_0, gamma_0, beta_0),
        (W1_1, b1_1, W2_1, b2_1, gamma_1, beta_1),
        (W1_2, b1_2, W2_2, b2_2, gamma_2, beta_2),
    ]
    e = edge_index.shape[1]
    e_pad = NS * (CPW0 + CPW1) * CHUNK
    assert e_pad >= e
    pad_cols = jnp.broadcast_to(
        jnp.array([[0], [DEAD]], jnp.int32), (2, e_pad - e))
    ei = jnp.concatenate([edge_index, pad_cols], axis=1)
    split = NS * CPW0 * CHUNK
    slot_pad = jnp.zeros((NS, CPW0 - CPW1, CHUNK), jnp.int32)

    def pools(flat):
        p0 = flat[:split].reshape(NS, CPW0, CHUNK)
        p1 = flat[split:].reshape(NS, CPW1, CHUNK)
        return jnp.concatenate(
            [p0, jnp.concatenate([p1, slot_pad], axis=1)], axis=0)
    src_w = pools(ei[0])
    dst_w = pools(ei[1])
    batch3 = batch.reshape(NBLK, 1, BLK)

    agg_fn = _make_agg()
    mlp_mid = _make_mlp(last=False)
    mlp_last = _make_mlp(last=True)

    z = x
    zs, gs = [], []
    for l in range(L):
        W1, b1, W2, b2, gamma, beta = params[l]
        scale = gamma / jnp.sqrt(1.0 + BN_EPS)
        w2f = W2 * scale[None, :]
        b2f = (b2 * scale + beta).reshape(1, D)
        b1r = b1.reshape(1, D)

        parts = agg_fn(z, src_w, dst_w)
        mlp = mlp_last if l == L - 1 else mlp_mid
        h, g = mlp(z, parts, batch3, W1, b1r, w2f, b2f)
        zs.append(h)
        gs.append(g)
        z = h

    return (jnp.concatenate(zs, axis=1), jnp.concatenate(gs, axis=1))


# speedup vs baseline: 1.9293x; 1.0146x over previous
import functools

import jax
import jax.numpy as jnp
from jax import lax
from jax.experimental import pallas as pl
from jax.experimental.pallas import tpu as pltpu
from jax.experimental.pallas import tpu_sc as plsc

N = 10000
D = 128
G = 64
L = 3
BN_EPS = 1e-5

NC = 2
NS = 16
NW = NC * NS
CHUNK = 128
BLK = 1000
NBLK = N // BLK
N_ACC = 10240
ROWS_PER_TILE = N_ACC // NS
ZCOPY = 128
CPW0 = 99
CPW1 = 58
DEAD = N


def _make_agg():
    mesh = plsc.VectorSubcoreMesh(core_axis_name="c", subcore_axis_name="s")

    @functools.partial(
        pl.kernel,
        out_type=jax.ShapeDtypeStruct((NC, N_ACC, D), jnp.float32),
        mesh=mesh,
        scratch_types=[
            pltpu.VMEM((CPW0, CHUNK), jnp.int32),
            pltpu.VMEM((CPW0, CHUNK), jnp.int32),
            pltpu.VMEM((CHUNK, D), jnp.float32),
            pltpu.VMEM_SHARED((N_ACC, D), jnp.float32),
            pltpu.SemaphoreType.DMA,
        ],
    )
    def agg(z_hbm, src_hbm, dst_hbm, out_hbm, src_v, dst_v, b0, acc_sh, gsem):
        c = lax.axis_index("c")
        s = lax.axis_index("s")
        row = c * NS + s
        n_c = lax.select(c == 0, jnp.int32(CPW0), jnp.int32(CPW1))

        def zrow(r, carry):
            for k in range(D // 16):
                b0[r, pl.ds(k * 16, 16)] = jnp.zeros((16,), jnp.float32)
            return carry
        lax.fori_loop(0, CHUNK, zrow, 0)
        base = s * ROWS_PER_TILE
        for k in range(ROWS_PER_TILE // ZCOPY):
            pltpu.sync_copy(b0.at[pl.ds(0, ZCOPY)],
                            acc_sh.at[pl.ds(base + k * ZCOPY, ZCOPY)])
        plsc.subcore_barrier()

        pltpu.sync_copy(src_hbm.at[row], src_v)
        pltpu.sync_copy(dst_hbm.at[row], dst_v)

        def body(j, carry):
            pltpu.async_copy(z_hbm.at[src_v.at[j]], b0, gsem).wait()
            pltpu.sync_copy(b0, acc_sh.at[dst_v.at[j]], add=True)
            return carry
        lax.fori_loop(0, n_c, body, 0)
        plsc.subcore_barrier()

        pltpu.sync_copy(acc_sh.at[pl.ds(base, ROWS_PER_TILE)],
                        out_hbm.at[c, pl.ds(base, ROWS_PER_TILE)])

    return agg


def _mlp_body(last: bool, z_ref, parts_ref, bt_ref, w1_ref, b1_ref, w2_ref,
              b2_ref, h_ref, g_ref):
    h = z_ref[...] + parts_ref[0] + parts_ref[1]
    h = jnp.maximum(
        jnp.dot(h, w1_ref[...], preferred_element_type=jnp.float32) + b1_ref[...],
        0.0)
    h = jnp.dot(h, w2_ref[...], preferred_element_type=jnp.float32) + b2_ref[...]
    if not last:
        h = jnp.maximum(h, 0.0)
    h_ref[...] = h

    b = bt_ref[0, 0, :]
    oh_t = (lax.broadcasted_iota(jnp.int32, (G, BLK), 0) == b[None, :]
            ).astype(jnp.float32)
    gpart = jnp.dot(oh_t, h, preferred_element_type=jnp.float32)

    @pl.when(pl.program_id(0) == 0)
    def _():
        g_ref[...] = jnp.zeros_like(g_ref)
    g_ref[...] += gpart


def _make_mlp(last: bool):
    return pl.pallas_call(
        functools.partial(_mlp_body, last),
        grid=(NBLK,),
        in_specs=[
            pl.BlockSpec((BLK, D), lambda i: (i, 0)),
            pl.BlockSpec((NC, BLK, D), lambda i: (0, i, 0)),
            pl.BlockSpec((1, 1, BLK), lambda i: (i, 0, 0)),
            pl.BlockSpec((D, D), lambda i: (0, 0)),
            pl.BlockSpec((1, D), lambda i: (0, 0)),
            pl.BlockSpec((D, D), lambda i: (0, 0)),
            pl.BlockSpec((1, D), lambda i: (0, 0)),
        ],
        out_specs=[
            pl.BlockSpec((BLK, D), lambda i: (i, 0)),
            pl.BlockSpec((G, D), lambda i: (0, 0)),
        ],
        out_shape=[
            jax.ShapeDtypeStruct((N, D), jnp.float32),
            jax.ShapeDtypeStruct((G, D), jnp.float32),
        ],
    )


def kernel(x, edge_index, batch,
           W1_0, b1_0, W2_0, b2_0, gamma_0, beta_0,
           W1_1, b1_1, W2_1, b2_1, gamma_1, beta_1,
           W1_2, b1_2, W2_2, b2_2, gamma_2, beta_2):
    params = [
        (W1_0, b1_0, W2_0, b2_0, gamma_0, beta_0),
        (W1_1, b1_1, W2_1, b2_1, gamma_1, beta_1),
        (W1_2, b1_2, W2_2, b2_2, gamma_2, beta_2),
    ]
    e = edge_index.shape[1]
    e_pad = NS * (CPW0 + CPW1) * CHUNK
    assert e_pad >= e
    pad_cols = jnp.broadcast_to(
        jnp.array([[0], [DEAD]], jnp.int32), (2, e_pad - e))
    ei = jnp.concatenate([edge_index, pad_cols], axis=1)
    split = NS * CPW0 * CHUNK
    slot_pad = jnp.zeros((NS, CPW0 - CPW1, CHUNK), jnp.int32)

    def pools(flat):
        p0 = flat[:split].reshape(NS, CPW0, CHUNK)
        p1 = flat[split:].reshape(NS, CPW1, CHUNK)
        return jnp.concatenate(
            [p0, jnp.concatenate([p1, slot_pad], axis=1)], axis=0)
    src_w = pools(ei[0])
    dst_w = pools(ei[1])
    batch3 = batch.reshape(NBLK, 1, BLK)

    agg_fn = _make_agg()
    mlp_mid = _make_mlp(last=False)
    mlp_last = _make_mlp(last=True)

    z = x
    zs, gs = [], []
    for l in range(L):
        W1, b1, W2, b2, gamma, beta = params[l]
        scale = gamma / jnp.sqrt(1.0 + BN_EPS)
        w2f = W2 * scale[None, :]
        b2f = (b2 * scale + beta).reshape(1, D)
        b1r = b1.reshape(1, D)

        parts = agg_fn(z, src_w, dst_w)
        mlp = mlp_last if l == L - 1 else mlp_mid
        h, g = mlp(z, parts, batch3, W1, b1r, w2f, b2f)
        zs.append(h)
        gs.append(g)
        z = h

    return (jnp.concatenate(zs, axis=1), jnp.concatenate(gs, axis=1))


# speedup vs baseline: 1.9626x; 1.0172x over previous
import functools

import jax
import jax.numpy as jnp
from jax import lax
from jax.experimental import pallas as pl
from jax.experimental.pallas import tpu as pltpu
from jax.experimental.pallas import tpu_sc as plsc

N = 10000
D = 128
G = 64
L = 3
BN_EPS = 1e-5

NC = 2
NS = 16
NW = NC * NS
CHUNK = 128
BLK = 1000
NBLK = N // BLK
N_ACC = 10240
ROWS_PER_TILE = N_ACC // NS
ZCOPY = 128
CPW0 = 97
CPW1 = 60
DEAD = N


def _make_agg():
    mesh = plsc.VectorSubcoreMesh(core_axis_name="c", subcore_axis_name="s")

    @functools.partial(
        pl.kernel,
        out_type=jax.ShapeDtypeStruct((NC, N_ACC, D), jnp.float32),
        mesh=mesh,
        scratch_types=[
            pltpu.VMEM((CPW0, CHUNK), jnp.int32),
            pltpu.VMEM((CPW0, CHUNK), jnp.int32),
            pltpu.VMEM((CHUNK, D), jnp.float32),
            pltpu.VMEM_SHARED((N_ACC, D), jnp.float32),
            pltpu.SemaphoreType.DMA,
        ],
    )
    def agg(z_hbm, src_hbm, dst_hbm, out_hbm, src_v, dst_v, b0, acc_sh, gsem):
        c = lax.axis_index("c")
        s = lax.axis_index("s")
        row = c * NS + s
        n_c = lax.select(c == 0, jnp.int32(CPW0), jnp.int32(CPW1))

        def zrow(r, carry):
            for k in range(D // 16):
                b0[r, pl.ds(k * 16, 16)] = jnp.zeros((16,), jnp.float32)
            return carry
        lax.fori_loop(0, CHUNK, zrow, 0)
        base = s * ROWS_PER_TILE
        for k in range(ROWS_PER_TILE // ZCOPY):
            pltpu.sync_copy(b0.at[pl.ds(0, ZCOPY)],
                            acc_sh.at[pl.ds(base + k * ZCOPY, ZCOPY)])
        plsc.subcore_barrier()

        pltpu.sync_copy(src_hbm.at[row], src_v)
        pltpu.sync_copy(dst_hbm.at[row], dst_v)

        def body(j, carry):
            pltpu.async_copy(z_hbm.at[src_v.at[j]], b0, gsem).wait()
            pltpu.sync_copy(b0, acc_sh.at[dst_v.at[j]], add=True)
            return carry
        lax.fori_loop(0, n_c, body, 0)
        plsc.subcore_barrier()

        pltpu.sync_copy(acc_sh.at[pl.ds(base, ROWS_PER_TILE)],
                        out_hbm.at[c, pl.ds(base, ROWS_PER_TILE)])

    return agg


def _mlp_body(last: bool, z_ref, parts_ref, bt_ref, w1_ref, b1_ref, w2_ref,
              b2_ref, h_ref, g_ref):
    h = z_ref[...] + parts_ref[0] + parts_ref[1]
    h = jnp.maximum(
        jnp.dot(h, w1_ref[...], preferred_element_type=jnp.float32) + b1_ref[...],
        0.0)
    h = jnp.dot(h, w2_ref[...], preferred_element_type=jnp.float32) + b2_ref[...]
    if not last:
        h = jnp.maximum(h, 0.0)
    h_ref[...] = h

    b = bt_ref[0, 0, :]
    oh_t = (lax.broadcasted_iota(jnp.int32, (G, BLK), 0) == b[None, :]
            ).astype(jnp.float32)
    gpart = jnp.dot(oh_t, h, preferred_element_type=jnp.float32)

    @pl.when(pl.program_id(0) == 0)
    def _():
        g_ref[...] = jnp.zeros_like(g_ref)
    g_ref[...] += gpart


def _make_mlp(last: bool):
    return pl.pallas_call(
        functools.partial(_mlp_body, last),
        grid=(NBLK,),
        in_specs=[
            pl.BlockSpec((BLK, D), lambda i: (i, 0)),
            pl.BlockSpec((NC, BLK, D), lambda i: (0, i, 0)),
            pl.BlockSpec((1, 1, BLK), lambda i: (i, 0, 0)),
            pl.BlockSpec((D, D), lambda i: (0, 0)),
            pl.BlockSpec((1, D), lambda i: (0, 0)),
            pl.BlockSpec((D, D), lambda i: (0, 0)),
            pl.BlockSpec((1, D), lambda i: (0, 0)),
        ],
        out_specs=[
            pl.BlockSpec((BLK, D), lambda i: (i, 0)),
            pl.BlockSpec((G, D), lambda i: (0, 0)),
        ],
        out_shape=[
            jax.ShapeDtypeStruct((N, D), jnp.float32),
            jax.ShapeDtypeStruct((G, D), jnp.float32),
        ],
    )


def kernel(x, edge_index, batch,
           W1_0, b1_0, W2_0, b2_0, gamma_0, beta_0,
           W1_1, b1_1, W2_1, b2_1, gamma_1, beta_1,
           W1_2, b1_2, W2_2, b2_2, gamma_2, beta_2):
    params = [
        (W1_0, b1_0, W2_0, b2_0, gamma_0, beta_0),
        (W1_1, b1_1, W2_1, b2_1, gamma_1, beta_1),
        (W1_2, b1_2, W2_2, b2_2, gamma_2, beta_2),
    ]
    e = edge_index.shape[1]
    e_pad = NS * (CPW0 + CPW1) * CHUNK
    assert e_pad >= e
    pad_cols = jnp.broadcast_to(
        jnp.array([[0], [DEAD]], jnp.int32), (2, e_pad - e))
    ei = jnp.concatenate([edge_index, pad_cols], axis=1)
    split = NS * CPW0 * CHUNK
    slot_pad = jnp.zeros((NS, CPW0 - CPW1, CHUNK), jnp.int32)

    def pools(flat):
        p0 = flat[:split].reshape(NS, CPW0, CHUNK)
        p1 = flat[split:].reshape(NS, CPW1, CHUNK)
        return jnp.concatenate(
            [p0, jnp.concatenate([p1, slot_pad], axis=1)], axis=0)
    src_w = pools(ei[0])
    dst_w = pools(ei[1])
    batch3 = batch.reshape(NBLK, 1, BLK)

    agg_fn = _make_agg()
    mlp_mid = _make_mlp(last=False)
    mlp_last = _make_mlp(last=True)

    z = x
    zs, gs = [], []
    for l in range(L):
        W1, b1, W2, b2, gamma, beta = params[l]
        scale = gamma / jnp.sqrt(1.0 + BN_EPS)
        w2f = W2 * scale[None, :]
        b2f = (b2 * scale + beta).reshape(1, D)
        b1r = b1.reshape(1, D)

        parts = agg_fn(z, src_w, dst_w)
        mlp = mlp_last if l == L - 1 else mlp_mid
        h, g = mlp(z, parts, batch3, W1, b1r, w2f, b2f)
        zs.append(h)
        gs.append(g)
        z = h

    return (jnp.concatenate(zs, axis=1), jnp.concatenate(gs, axis=1))
